# baseline (speedup 1.0000x reference)
import dataclasses
import functools

import jax
import jax.numpy as jnp
from jax import lax
from jax.experimental import pallas as pl
from jax.experimental.pallas import tpu as pltpu
from jax.experimental.pallas import tpu_sc as plsc

B, N, C, NCLS = 64, 900, 256, 80
K = 12
BETA = 0.08
L = 16
NPAD = 1024
NCHUNK = 57
NEG = -3.0e38


def _quality_body(s_ref, q_ref):
    m = jnp.max(s_ref[...], axis=1)
    pad = jnp.full((m.shape[0], NPAD - N), NEG, m.dtype)
    q_ref[...] = jnp.concatenate([m, pad], axis=-1)


def _quality(scores_cn):
    blk = 16
    return pl.pallas_call(
        _quality_body,
        grid=(B // blk,),
        in_specs=[pl.BlockSpec((blk, NCLS, N), lambda i: (i, 0, 0))],
        out_specs=pl.BlockSpec((blk, NPAD), lambda i: (i, 0)),
        out_shape=jax.ShapeDtypeStruct((B, NPAD), jnp.float32),
    )(scores_cn)


def _sc_body(q_hbm, feat_hbm, out_hbm, qrow, idx0, idx1, rows0, rows1,
             pooled, sem0, sem1):
    wid = lax.axis_index("s") * 2 + lax.axis_index("c")
    lane = lax.iota(jnp.int32, L)

    def select_row(b, idxbuf):
        pltpu.sync_copy(q_hbm.at[b], qrow)

        def merge_chunk(i, carry):
            best, thr = carry
            off = pl.multiple_of(i * L, L)
            v = qrow[pl.ds(off, L)]
            m = jnp.max(v)

            def do_merge(c):
                best, _ = c
                v_desc = jnp.sort(v)[::-1]
                nb = jnp.sort(jnp.maximum(best, v_desc))
                return nb, jnp.min(nb)

            return lax.cond(m > thr, do_merge, lambda c: c, carry)

        best, _ = lax.fori_loop(
            0, NCHUNK, merge_chunk,
            (jnp.full((L,), NEG, jnp.float32), jnp.float32(NEG)))
        vk = jnp.max(jnp.where(lane == (L - K), best, NEG))
        c_gt = jnp.sum((best > vk).astype(jnp.int32))
        need_eq = K - c_gt

        def select_chunk(i, carry):
            nsel, neq = carry
            off = pl.multiple_of(i * L, L)
            v = qrow[pl.ds(off, L)]
            m = jnp.max(v)

            def work(c):
                nsel, neq = c
                gtm = v > vk
                eqm = v == vk
                eq_pref = jnp.cumsum(eqm.astype(jnp.int32))
                sel = gtm | (eqm & ((neq + eq_pref) <= need_eq))
                sel_pref = jnp.cumsum(sel.astype(jnp.int32))
                pos = nsel + sel_pref - 1
                rec = (i * L + lane) * 8 + (b // 8)
                plsc.store_scatter(idxbuf, [pos], rec, mask=sel)
                return (nsel + jnp.sum(sel.astype(jnp.int32)),
                        neq + jnp.sum(eqm.astype(jnp.int32)))

            return lax.cond((m >= vk) & (nsel < K), work, lambda c: c, carry)

        lax.fori_loop(0, NCHUNK, select_chunk,
                      (jnp.int32(0), jnp.int32(0)))

    def pool_row(b, rows):
        b_lo = b % 8
        inv_k = jnp.float32(1.0 / K)
        for ct in range(2):
            for cb in range(128 // L):
                acc = rows[0, ct, b_lo, pl.ds(cb * L, L)]
                for r in range(1, K):
                    acc = acc + rows[r, ct, b_lo, pl.ds(cb * L, L)]
                pooled[pl.ds(ct * 128 + cb * L, L)] = acc * inv_k
        pltpu.sync_copy(pooled, out_hbm.at[b])

    b0 = wid * 2
    b1 = b0 + 1
    select_row(b0, idx0)
    cp0 = pltpu.async_copy(feat_hbm.at[idx0], rows0, sem0)
    select_row(b1, idx1)
    cp1 = pltpu.async_copy(feat_hbm.at[idx1], rows1, sem1)
    cp0.wait()
    pool_row(b0, rows0)
    cp1.wait()
    pool_row(b1, rows1)


def _topk_pool(q, feat2d):
    mesh = plsc.VectorSubcoreMesh(core_axis_name="c", subcore_axis_name="s")
    cp = pltpu.CompilerParams()
    if "needs_layout_passes" in pltpu.CompilerParams.__dataclass_fields__:
        cp = dataclasses.replace(cp, needs_layout_passes=False)
    fn = pl.kernel(
        _sc_body,
        mesh=mesh,
        compiler_params=cp,
        out_type=jax.ShapeDtypeStruct((B, C), jnp.float32),
        scratch_types=[
            pltpu.VMEM((NPAD,), jnp.float32),
            pltpu.VMEM((K,), jnp.int32),
            pltpu.VMEM((K,), jnp.int32),
            pltpu.VMEM((K, 2, 8, 128), jnp.float32),
            pltpu.VMEM((K, 2, 8, 128), jnp.float32),
            pltpu.VMEM((C,), jnp.float32),
            pltpu.SemaphoreType.DMA,
            pltpu.SemaphoreType.DMA,
        ],
    )
    return fn(q, feat2d)


_DN_T = (((1,), (1,)), ((), ()))


def _gru_body(pooled_ref, h_ref, wg_ref, bg_ref, wih_ref, whh_ref, bih_ref,
              bhh_ref, lnw_ref, lnb_ref, scale_ref):
    pooled = pooled_ref[...]
    h = h_ref[...]
    gi = lax.dot_general(pooled, wih_ref[...], _DN_T,
                         preferred_element_type=jnp.float32) + bih_ref[...]
    gh = lax.dot_general(h, whh_ref[...], _DN_T,
                         preferred_element_type=jnp.float32) + bhh_ref[...]
    i_r, i_z, i_n = gi[:, :C], gi[:, C:2 * C], gi[:, 2 * C:]
    h_r, h_z, h_n = gh[:, :C], gh[:, C:2 * C], gh[:, 2 * C:]
    r = jax.nn.sigmoid(i_r + h_r)
    z = jax.nn.sigmoid(i_z + h_z)
    ng = jnp.tanh(i_n + r * h_n)
    hnew = (1.0 - z) * ng + z * h
    mu = jnp.mean(hnew, axis=-1, keepdims=True)
    var = jnp.mean((hnew - mu) ** 2, axis=-1, keepdims=True)
    mem = (hnew - mu) / jnp.sqrt(var + 1e-5) * lnw_ref[...] + lnb_ref[...]
    gate = jax.nn.sigmoid(
        lax.dot_general(mem, wg_ref[...], _DN_T,
                        preferred_element_type=jnp.float32) + bg_ref[...])
    scale_ref[...] = 1.0 + BETA * gate


def _gru_gate(pooled, h, wg_t, b_gate, wih_t, whh_t, b_ih, b_hh, ln_w, ln_b):
    return pl.pallas_call(
        _gru_body,
        out_shape=jax.ShapeDtypeStruct((B, C), jnp.float32),
    )(pooled, h, wg_t, b_gate, wih_t, whh_t, b_ih, b_hh, ln_w, ln_b)


def _apply_body(qf_ref, sc_ref, out_ref):
    out_ref[...] = qf_ref[...] * sc_ref[...]


def _apply(qf_t, scale3):
    blk = 100
    return pl.pallas_call(
        _apply_body,
        grid=(N // blk,),
        in_specs=[pl.BlockSpec((blk, B, C), lambda i: (i, 0, 0)),
                  pl.BlockSpec((1, B, C), lambda i: (0, 0, 0))],
        out_specs=pl.BlockSpec((blk, B, C), lambda i: (i, 0, 0)),
        out_shape=jax.ShapeDtypeStruct((N, B, C), jnp.float32),
    )(qf_t, scale3)


def kernel(query_feat, scores, prev_memory, W_gate, b_gate, W_ih, W_hh,
           b_ih, b_hh, ln_w, ln_b):
    scores_cn = scores.transpose(0, 2, 1)
    qf_t = query_feat.transpose(1, 0, 2)
    q = _quality(scores_cn)
    feat_rec = (query_feat.transpose(1, 0, 2)
                .reshape(N, 8, 8, 2, 128)
                .transpose(0, 1, 3, 2, 4)
                .reshape(N * 8, 2, 8, 128))
    pooled = _topk_pool(q, feat_rec)
    scale = _gru_gate(pooled, prev_memory, W_gate, b_gate, W_ih, W_hh,
                      b_ih, b_hh, ln_w, ln_b)
    out_t = _apply(qf_t, scale.reshape(1, B, C))
    return out_t.transpose(1, 0, 2)


# speedup vs baseline: 1.0148x; 1.0148x over previous
import dataclasses
import functools

import jax
import jax.numpy as jnp
from jax import lax
from jax.experimental import pallas as pl
from jax.experimental.pallas import tpu as pltpu
from jax.experimental.pallas import tpu_sc as plsc

B, N, C, NCLS = 64, 900, 256, 80
K = 12
BETA = 0.08
L = 16
NPAD = 1024
NCHUNK = 57
NEG = -3.0e38


def _quality_body(s_ref, q_ref):
    m = jnp.max(s_ref[...], axis=1)
    pad = jnp.full((m.shape[0], NPAD - N), NEG, m.dtype)
    q_ref[...] = jnp.concatenate([m, pad], axis=-1)


def _quality(scores_cn):
    blk = 16
    return pl.pallas_call(
        _quality_body,
        grid=(B // blk,),
        in_specs=[pl.BlockSpec((blk, NCLS, N), lambda i: (i, 0, 0))],
        out_specs=pl.BlockSpec((blk, NPAD), lambda i: (i, 0)),
        out_shape=jax.ShapeDtypeStruct((B, NPAD), jnp.float32),
    )(scores_cn)


def _sc_body(q_hbm, feat_hbm, out_hbm, qrow, idx0, idx1, rows0, rows1,
             pooled, sem0, sem1):
    wid = lax.axis_index("s") * 2 + lax.axis_index("c")
    lane = lax.iota(jnp.int32, L)

    def select_row(b, idxbuf):
        pltpu.sync_copy(q_hbm.at[b], qrow)

        svs = [jnp.sort(qrow[pl.ds(i * L, L)]) for i in range(NCHUNK)]
        while len(svs) > 1:
            nxt = [jnp.sort(jnp.maximum(svs[j], svs[j + 1][::-1]))
                   for j in range(0, len(svs) - 1, 2)]
            if len(svs) % 2:
                nxt.append(svs[-1])
            svs = nxt
        best = svs[0]
        vk = jnp.max(jnp.where(lane == (L - K), best, NEG))
        c_gt = jnp.sum((best > vk).astype(jnp.int32))
        need_eq = K - c_gt

        nsel = jnp.int32(0)
        neq = jnp.int32(0)
        for i in range(NCHUNK):
            v = qrow[pl.ds(i * L, L)]
            gtm = v > vk
            eqm = v == vk
            eq_pref = jnp.cumsum(eqm.astype(jnp.int32))
            sel = gtm | (eqm & ((neq + eq_pref) <= need_eq))
            sel_pref = jnp.cumsum(sel.astype(jnp.int32))
            pos = nsel + sel_pref - 1
            rec = (i * L + lane) * 8 + (b // 8)
            plsc.store_scatter(idxbuf, [pos], rec, mask=sel)
            nsel = nsel + jnp.sum(sel.astype(jnp.int32))
            neq = neq + jnp.sum(eqm.astype(jnp.int32))

    def pool_row(b, rows):
        b_lo = b % 8
        inv_k = jnp.float32(1.0 / K)
        for ct in range(2):
            for cb in range(128 // L):
                acc = rows[0, ct, b_lo, pl.ds(cb * L, L)]
                for r in range(1, K):
                    acc = acc + rows[r, ct, b_lo, pl.ds(cb * L, L)]
                pooled[pl.ds(ct * 128 + cb * L, L)] = acc * inv_k
        pltpu.sync_copy(pooled, out_hbm.at[b])

    b0 = wid * 2
    b1 = b0 + 1
    select_row(b0, idx0)
    cp0 = pltpu.async_copy(feat_hbm.at[idx0], rows0, sem0)
    select_row(b1, idx1)
    cp1 = pltpu.async_copy(feat_hbm.at[idx1], rows1, sem1)
    cp0.wait()
    pool_row(b0, rows0)
    cp1.wait()
    pool_row(b1, rows1)


def _topk_pool(q, feat2d):
    mesh = plsc.VectorSubcoreMesh(core_axis_name="c", subcore_axis_name="s")
    cp = pltpu.CompilerParams()
    if "needs_layout_passes" in pltpu.CompilerParams.__dataclass_fields__:
        cp = dataclasses.replace(cp, needs_layout_passes=False)
    fn = pl.kernel(
        _sc_body,
        mesh=mesh,
        compiler_params=cp,
        out_type=jax.ShapeDtypeStruct((B, C), jnp.float32),
        scratch_types=[
            pltpu.VMEM((NPAD,), jnp.float32),
            pltpu.VMEM((K,), jnp.int32),
            pltpu.VMEM((K,), jnp.int32),
            pltpu.VMEM((K, 2, 8, 128), jnp.float32),
            pltpu.VMEM((K, 2, 8, 128), jnp.float32),
            pltpu.VMEM((C,), jnp.float32),
            pltpu.SemaphoreType.DMA,
            pltpu.SemaphoreType.DMA,
        ],
    )
    return fn(q, feat2d)


_DN_T = (((1,), (1,)), ((), ()))


def _gru_body(pooled_ref, h_ref, wg_ref, bg_ref, wih_ref, whh_ref, bih_ref,
              bhh_ref, lnw_ref, lnb_ref, scale_ref):
    pooled = pooled_ref[...]
    h = h_ref[...]
    gi = lax.dot_general(pooled, wih_ref[...], _DN_T,
                         preferred_element_type=jnp.float32) + bih_ref[...]
    gh = lax.dot_general(h, whh_ref[...], _DN_T,
                         preferred_element_type=jnp.float32) + bhh_ref[...]
    i_r, i_z, i_n = gi[:, :C], gi[:, C:2 * C], gi[:, 2 * C:]
    h_r, h_z, h_n = gh[:, :C], gh[:, C:2 * C], gh[:, 2 * C:]
    r = jax.nn.sigmoid(i_r + h_r)
    z = jax.nn.sigmoid(i_z + h_z)
    ng = jnp.tanh(i_n + r * h_n)
    hnew = (1.0 - z) * ng + z * h
    mu = jnp.mean(hnew, axis=-1, keepdims=True)
    var = jnp.mean((hnew - mu) ** 2, axis=-1, keepdims=True)
    mem = (hnew - mu) / jnp.sqrt(var + 1e-5) * lnw_ref[...] + lnb_ref[...]
    gate = jax.nn.sigmoid(
        lax.dot_general(mem, wg_ref[...], _DN_T,
                        preferred_element_type=jnp.float32) + bg_ref[...])
    scale_ref[...] = 1.0 + BETA * gate


def _gru_gate(pooled, h, wg_t, b_gate, wih_t, whh_t, b_ih, b_hh, ln_w, ln_b):
    return pl.pallas_call(
        _gru_body,
        out_shape=jax.ShapeDtypeStruct((B, C), jnp.float32),
    )(pooled, h, wg_t, b_gate, wih_t, whh_t, b_ih, b_hh, ln_w, ln_b)


def _apply_body(qf_ref, sc_ref, out_ref):
    out_ref[...] = qf_ref[...] * sc_ref[...]


def _apply(qf_t, scale3):
    blk = 100
    return pl.pallas_call(
        _apply_body,
        grid=(N // blk,),
        in_specs=[pl.BlockSpec((blk, B, C), lambda i: (i, 0, 0)),
                  pl.BlockSpec((1, B, C), lambda i: (0, 0, 0))],
        out_specs=pl.BlockSpec((blk, B, C), lambda i: (i, 0, 0)),
        out_shape=jax.ShapeDtypeStruct((N, B, C), jnp.float32),
    )(qf_t, scale3)


def kernel(query_feat, scores, prev_memory, W_gate, b_gate, W_ih, W_hh,
           b_ih, b_hh, ln_w, ln_b):
    scores_cn = scores.transpose(0, 2, 1)
    qf_t = query_feat.transpose(1, 0, 2)
    q = _quality(scores_cn)
    feat_rec = (query_feat.transpose(1, 0, 2)
                .reshape(N, 8, 8, 2, 128)
                .transpose(0, 1, 3, 2, 4)
                .reshape(N * 8, 2, 8, 128))
    pooled = _topk_pool(q, feat_rec)
    scale = _gru_gate(pooled, prev_memory, W_gate, b_gate, W_ih, W_hh,
                      b_ih, b_hh, ln_w, ln_b)
    out_t = _apply(qf_t, scale.reshape(1, B, C))
    return out_t.transpose(1, 0, 2)


# speedup vs baseline: 1.0444x; 1.0291x over previous
import dataclasses
import functools

import jax
import jax.numpy as jnp
from jax import lax
from jax.experimental import pallas as pl
from jax.experimental.pallas import tpu as pltpu
from jax.experimental.pallas import tpu_sc as plsc

B, N, C, NCLS = 64, 900, 256, 80
K = 12
BETA = 0.08
L = 16
NPAD = 1024
NCHUNK = 57
NEG = -3.0e38


def _quality_body(s_ref, q_ref):
    m = jnp.max(s_ref[...], axis=1)
    pad = jnp.full((m.shape[0], NPAD - N), NEG, m.dtype)
    q_ref[...] = jnp.concatenate([m, pad], axis=-1)


def _quality(scores_cn):
    blk = 16
    return pl.pallas_call(
        _quality_body,
        grid=(B // blk,),
        in_specs=[pl.BlockSpec((blk, NCLS, N), lambda i: (i, 0, 0))],
        out_specs=pl.BlockSpec((blk, NPAD), lambda i: (i, 0)),
        out_shape=jax.ShapeDtypeStruct((B, NPAD), jnp.float32),
    )(scores_cn)


def _sc_body(q_hbm, feat_hbm, out_hbm, qrow, schunks, idxbuf, rows,
             pooled, sem):
    wid = lax.axis_index("s") * 2 + lax.axis_index("c")
    lane = lax.iota(jnp.int32, L)
    zero16 = jnp.zeros((L,), jnp.int32)
    inv_k = jnp.float32(1.0 / K)

    @pl.loop(0, 2)
    def _(t):
        b = wid * 2 + t
        pltpu.sync_copy(q_hbm.at[b], qrow)

        def sort_chunk(i, c):
            off = pl.multiple_of(i * L, L)
            schunks[pl.ds(off, L)] = jnp.sort(qrow[pl.ds(off, L)])
            return c

        lax.fori_loop(0, NCHUNK, sort_chunk, 0)

        def merge(i, best):
            off = pl.multiple_of(i * L, L)
            return jnp.sort(jnp.maximum(best, schunks[pl.ds(off, L)][::-1]))

        best = lax.fori_loop(0, NCHUNK, merge,
                             jnp.full((L,), NEG, jnp.float32))
        vk = jnp.max(jnp.where(lane == (L - K), best, NEG))
        c_gt = plsc.all_reduce_population_count(best > vk)
        need_eq = K - c_gt

        def select_chunk(i, carry):
            nsel, neq = carry
            off = pl.multiple_of(i * L, L)
            v = qrow[pl.ds(off, L)]
            gtm = v > vk
            eqm = v == vk
            eq_pref = jnp.cumsum(eqm.astype(jnp.int32))
            sel = gtm | (eqm & ((neq + eq_pref) <= need_eq))
            sel_pref = jnp.cumsum(sel.astype(jnp.int32))
            pos = nsel + sel_pref - 1
            rec = (i * L + lane) * 8 + (b // 8)
            plsc.store_scatter(idxbuf, [pos], rec, mask=sel)
            return (nsel + plsc.all_reduce_population_count(sel),
                    neq + plsc.all_reduce_population_count(eqm))

        lax.fori_loop(0, NCHUNK, select_chunk, (zero16, zero16))

        pltpu.async_copy(feat_hbm.at[idxbuf], rows, sem).wait()
        b_lo = b % 8

        def cchunk(cc, c):
            ct = cc // 8
            off = pl.multiple_of((cc % 8) * L, L)
            acc = rows[0, ct, b_lo, pl.ds(off, L)]
            for r in range(1, K):
                acc = acc + rows[r, ct, b_lo, pl.ds(off, L)]
            pooled[pl.ds(pl.multiple_of(cc * L, L), L)] = acc * inv_k
            return c

        lax.fori_loop(0, 2 * (128 // L), cchunk, 0)
        pltpu.sync_copy(pooled, out_hbm.at[b])


def _topk_pool(q, feat2d):
    mesh = plsc.VectorSubcoreMesh(core_axis_name="c", subcore_axis_name="s")
    cp = pltpu.CompilerParams()
    if "needs_layout_passes" in pltpu.CompilerParams.__dataclass_fields__:
        cp = dataclasses.replace(cp, needs_layout_passes=False)
    fn = pl.kernel(
        _sc_body,
        mesh=mesh,
        compiler_params=cp,
        out_type=jax.ShapeDtypeStruct((B, C), jnp.float32),
        scratch_types=[
            pltpu.VMEM((NPAD,), jnp.float32),
            pltpu.VMEM((NPAD,), jnp.float32),
            pltpu.VMEM((K,), jnp.int32),
            pltpu.VMEM((K, 2, 8, 128), jnp.float32),
            pltpu.VMEM((C,), jnp.float32),
            pltpu.SemaphoreType.DMA,
        ],
    )
    return fn(q, feat2d)


_DN_T = (((1,), (1,)), ((), ()))


def _gru_body(pooled_ref, h_ref, wg_ref, bg_ref, wih_ref, whh_ref, bih_ref,
              bhh_ref, lnw_ref, lnb_ref, scale_ref):
    pooled = pooled_ref[...]
    h = h_ref[...]
    gi = lax.dot_general(pooled, wih_ref[...], _DN_T,
                         preferred_element_type=jnp.float32) + bih_ref[...]
    gh = lax.dot_general(h, whh_ref[...], _DN_T,
                         preferred_element_type=jnp.float32) + bhh_ref[...]
    i_r, i_z, i_n = gi[:, :C], gi[:, C:2 * C], gi[:, 2 * C:]
    h_r, h_z, h_n = gh[:, :C], gh[:, C:2 * C], gh[:, 2 * C:]
    r = jax.nn.sigmoid(i_r + h_r)
    z = jax.nn.sigmoid(i_z + h_z)
    ng = jnp.tanh(i_n + r * h_n)
    hnew = (1.0 - z) * ng + z * h
    mu = jnp.mean(hnew, axis=-1, keepdims=True)
    var = jnp.mean((hnew - mu) ** 2, axis=-1, keepdims=True)
    mem = (hnew - mu) / jnp.sqrt(var + 1e-5) * lnw_ref[...] + lnb_ref[...]
    gate = jax.nn.sigmoid(
        lax.dot_general(mem, wg_ref[...], _DN_T,
                        preferred_element_type=jnp.float32) + bg_ref[...])
    scale_ref[...] = 1.0 + BETA * gate


def _gru_gate(pooled, h, wg_t, b_gate, wih_t, whh_t, b_ih, b_hh, ln_w, ln_b):
    return pl.pallas_call(
        _gru_body,
        out_shape=jax.ShapeDtypeStruct((B, C), jnp.float32),
    )(pooled, h, wg_t, b_gate, wih_t, whh_t, b_ih, b_hh, ln_w, ln_b)


def _apply_body(qf_ref, sc_ref, out_ref):
    out_ref[...] = qf_ref[...] * sc_ref[...]


def _apply(qf_t, scale3):
    blk = 100
    return pl.pallas_call(
        _apply_body,
        grid=(N // blk,),
        in_specs=[pl.BlockSpec((blk, B, C), lambda i: (i, 0, 0)),
                  pl.BlockSpec((1, B, C), lambda i: (0, 0, 0))],
        out_specs=pl.BlockSpec((blk, B, C), lambda i: (i, 0, 0)),
        out_shape=jax.ShapeDtypeStruct((N, B, C), jnp.float32),
    )(qf_t, scale3)


def kernel(query_feat, scores, prev_memory, W_gate, b_gate, W_ih, W_hh,
           b_ih, b_hh, ln_w, ln_b):
    scores_cn = scores.transpose(0, 2, 1)
    qf_t = query_feat.transpose(1, 0, 2)
    q = _quality(scores_cn)
    feat_rec = (query_feat.transpose(1, 0, 2)
                .reshape(N, 8, 8, 2, 128)
                .transpose(0, 1, 3, 2, 4)
                .reshape(N * 8, 2, 8, 128))
    pooled = _topk_pool(q, feat_rec)
    scale = _gru_gate(pooled, prev_memory, W_gate, b_gate, W_ih, W_hh,
                      b_ih, b_hh, ln_w, ln_b)
    out_t = _apply(qf_t, scale.reshape(1, B, C))
    return out_t.transpose(1, 0, 2)
